# Initial kernel scaffold; baseline (speedup 1.0000x reference)
import jax
import jax.numpy as jnp
from jax.experimental import pallas as pl


def kernel(vectors, node_feats, radial_embedding, senders, receivers, W1, W2, W3, W4):
    raise NotImplementedError("write your pallas kernel here")


# speedup vs baseline: 2.2735x; 2.2735x over previous
import math

import jax
import jax.numpy as jnp
import numpy as np
from jax import lax
from jax.experimental import pallas as pl
from jax.experimental.pallas import tpu as pltpu
from jax.experimental.pallas import tpu_sc as plsc

N = 10000
E = 320000
NS = 16
K = 80
BE = 512

ROWS_PER_TILE = N // NS
E_PER_TILE = E // NS
NCHUNK = E_PER_TILE // K
ZROWS = 25

_SQRT3 = math.sqrt(3.0)
_INV_SQRT_NEIGH = 1.0 / math.sqrt(32.0)

_COLPERM = np.concatenate([
    np.arange(64),
    64 + (np.arange(96) % 3) * 32 + np.arange(96) // 3,
    160 + (np.arange(96) % 3) * 32 + np.arange(96) // 3,
]).astype(np.int32)


def _tc_body(r_ref, v_ref, w1, w2, w3, w4, aux0_ref, aux1_ref):
    r = r_ref[...]
    v = v_ref[...]
    h = jax.nn.silu(jnp.dot(r, w1[...], preferred_element_type=jnp.float32))
    h = jax.nn.silu(jnp.dot(h, w2[...], preferred_element_type=jnp.float32))
    h = jax.nn.silu(jnp.dot(h, w3[...], preferred_element_type=jnp.float32))
    mix = jnp.dot(h, w4[...], preferred_element_type=jnp.float32) * _INV_SQRT_NEIGH
    rnorm = jnp.sqrt(jnp.sum(v * v, axis=1, keepdims=True) + 1e-12)
    y = v * (_SQRT3 / rnorm)
    a_s = mix[:, 0:32]
    a_t0 = mix[:, 32:64] * (1.0 / _SQRT3)
    a_v = mix[:, 64:96]
    a_t1 = mix[:, 96:128]
    ones16 = jnp.ones((1, 16), jnp.float32)
    aux0_ref[:, 0:32] = a_s
    aux0_ref[:, 32:64] = a_t1 * y[:, 0:1]
    aux0_ref[:, 64:96] = a_t1 * y[:, 1:2]
    aux0_ref[:, 96:128] = a_t1 * y[:, 2:3]
    aux1_ref[:, 0:32] = a_v
    aux1_ref[:, 32:64] = a_t0
    aux1_ref[:, 64:80] = y[:, 0:1] * ones16
    aux1_ref[:, 80:96] = y[:, 1:2] * ones16
    aux1_ref[:, 96:112] = y[:, 2:3] * ones16
    aux1_ref[:, 112:128] = jnp.zeros((r.shape[0], 16), jnp.float32)


_tc_aux = pl.pallas_call(
    _tc_body,
    grid=(E // BE,),
    in_specs=[
        pl.BlockSpec((BE, 8), lambda i: (i, 0)),
        pl.BlockSpec((BE, 3), lambda i: (i, 0)),
        pl.BlockSpec((8, 64), lambda i: (0, 0)),
        pl.BlockSpec((64, 64), lambda i: (0, 0)),
        pl.BlockSpec((64, 64), lambda i: (0, 0)),
        pl.BlockSpec((64, 128), lambda i: (0, 0)),
    ],
    out_specs=[
        pl.BlockSpec((BE, 128), lambda i: (i, 0)),
        pl.BlockSpec((BE, 128), lambda i: (i, 0)),
    ],
    out_shape=[
        jax.ShapeDtypeStruct((E, 128), jnp.float32),
        jax.ShapeDtypeStruct((E, 128), jnp.float32),
    ],
)


def _sc_body(nfs, nfv, snd, rcv, aux0, aux1, out, acc, sidx, ridx, gs, gv,
             ab0, ab1, msg, zbuf, sem):
    c = lax.axis_index("c")
    s = lax.axis_index("s")
    r0 = s * ROWS_PER_TILE
    e0 = s * E_PER_TILE

    zeros16 = jnp.zeros((16,), jnp.float32)

    def _zrow(i, carry):
        for t in range(8):
            zbuf[i, pl.ds(16 * t, 16)] = zeros16
        return carry
    lax.fori_loop(0, ZROWS, _zrow, 0)
    for rep in range(ROWS_PER_TILE // ZROWS):
        pltpu.sync_copy(zbuf, acc.at[pl.ds(r0 + rep * ZROWS, ZROWS)])
    plsc.subcore_barrier()

    @pl.when(c == 0)
    def _core0():
        def chunk(j, carry):
            base = e0 + j * K
            pltpu.sync_copy(snd.at[pl.ds(base, K)], sidx)
            pltpu.sync_copy(rcv.at[pl.ds(base, K)], ridx)
            cp = pltpu.async_copy(nfs.at[sidx], gs, sem)
            pltpu.sync_copy(aux0.at[pl.ds(base, K)], ab0)
            cp.wait()

            def edge(k, ecarry):
                g0 = gs[k, pl.ds(0, 16)]
                g1 = gs[k, pl.ds(16, 16)]
                msg[k, pl.ds(0, 16)] = g0 * ab0[k, pl.ds(0, 16)]
                msg[k, pl.ds(16, 16)] = g1 * ab0[k, pl.ds(16, 16)]
                for t in range(6):
                    gh = g0 if t % 2 == 0 else g1
                    msg[k, pl.ds(32 + 16 * t, 16)] = gh * ab0[k, pl.ds(32 + 16 * t, 16)]
                return ecarry
            lax.fori_loop(0, K, edge, 0)
            pltpu.sync_copy(msg, acc.at[ridx], add=True)
            return carry
        lax.fori_loop(0, NCHUNK, chunk, 0)
        plsc.subcore_barrier()
        pltpu.sync_copy(acc.at[pl.ds(r0, ROWS_PER_TILE), pl.ds(0, 32)],
                        out.at[pl.ds(r0, ROWS_PER_TILE), pl.ds(0, 32)])
        pltpu.sync_copy(acc.at[pl.ds(r0, ROWS_PER_TILE), pl.ds(32, 96)],
                        out.at[pl.ds(r0, ROWS_PER_TILE), pl.ds(160, 96)])

    @pl.when(c == 1)
    def _core1():
        def chunk(j, carry):
            base = e0 + j * K
            pltpu.sync_copy(snd.at[pl.ds(base, K)], sidx)
            pltpu.sync_copy(rcv.at[pl.ds(base, K)], ridx)
            cp = pltpu.async_copy(nfv.at[sidx], gv, sem)
            pltpu.sync_copy(aux1.at[pl.ds(base, K)], ab1)
            cp.wait()

            def edge(k, ecarry):
                av0 = ab1[k, pl.ds(0, 16)]
                av1 = ab1[k, pl.ds(16, 16)]
                at0 = ab1[k, pl.ds(32, 16)]
                at1 = ab1[k, pl.ds(48, 16)]
                yb0 = ab1[k, pl.ds(64, 16)]
                yb1 = ab1[k, pl.ds(80, 16)]
                yb2 = ab1[k, pl.ds(96, 16)]
                gx0 = gv[k, pl.ds(0, 16)]
                gx1 = gv[k, pl.ds(16, 16)]
                gy0 = gv[k, pl.ds(32, 16)]
                gy1 = gv[k, pl.ds(48, 16)]
                gz0 = gv[k, pl.ds(64, 16)]
                gz1 = gv[k, pl.ds(80, 16)]
                msg[k, pl.ds(0, 16)] = (gx0 * yb0 + gy0 * yb1 + gz0 * yb2) * at0
                msg[k, pl.ds(16, 16)] = (gx1 * yb0 + gy1 * yb1 + gz1 * yb2) * at1
                msg[k, pl.ds(32, 16)] = gx0 * av0
                msg[k, pl.ds(48, 16)] = gx1 * av1
                msg[k, pl.ds(64, 16)] = gy0 * av0
                msg[k, pl.ds(80, 16)] = gy1 * av1
                msg[k, pl.ds(96, 16)] = gz0 * av0
                msg[k, pl.ds(112, 16)] = gz1 * av1
                return ecarry
            lax.fori_loop(0, K, edge, 0)
            pltpu.sync_copy(msg, acc.at[ridx], add=True)
            return carry
        lax.fori_loop(0, NCHUNK, chunk, 0)
        plsc.subcore_barrier()
        pltpu.sync_copy(acc.at[pl.ds(r0, ROWS_PER_TILE), pl.ds(0, 128)],
                        out.at[pl.ds(r0, ROWS_PER_TILE), pl.ds(32, 128)])


def _make_sc():
    mesh = plsc.VectorSubcoreMesh(core_axis_name="c", subcore_axis_name="s")
    return pl.kernel(
        _sc_body,
        out_type=jax.ShapeDtypeStruct((N, 256), jnp.float32),
        mesh=mesh,
        compiler_params=pltpu.CompilerParams(use_tc_tiling_on_sc=False),
        scratch_types=[
            pltpu.VMEM_SHARED((N, 128), jnp.float32),
            pltpu.VMEM((K,), jnp.int32),
            pltpu.VMEM((K,), jnp.int32),
            pltpu.VMEM((K, 32), jnp.float32),
            pltpu.VMEM((K, 96), jnp.float32),
            pltpu.VMEM((K, 128), jnp.float32),
            pltpu.VMEM((K, 128), jnp.float32),
            pltpu.VMEM((K, 128), jnp.float32),
            pltpu.VMEM((ZROWS, 128), jnp.float32),
            pltpu.SemaphoreType.DMA,
        ],
    )


def kernel(vectors, node_feats, radial_embedding, senders, receivers,
           W1, W2, W3, W4):
    w1 = W1 * (1.0 / math.sqrt(W1.shape[0]))
    w2 = W2 * (1.0 / math.sqrt(W2.shape[0]))
    w3 = W3 * (1.0 / math.sqrt(W3.shape[0]))
    w4 = W4 * (1.0 / math.sqrt(W4.shape[0]))
    aux0, aux1 = _tc_aux(radial_embedding, vectors, w1, w2, w3, w4)
    nf_s = node_feats[:, 0:32]
    nf_v = node_feats[:, 32:128].reshape(N, 32, 3).transpose(0, 2, 1).reshape(N, 96)
    snd = senders.astype(jnp.int32)
    rcv = receivers.astype(jnp.int32)
    outp = _make_sc()(nf_s, nf_v, snd, rcv, aux0, aux1)
    return jnp.take(outp, jnp.asarray(_COLPERM), axis=1)


# speedup vs baseline: 2.5943x; 1.1411x over previous
import math

import jax
import jax.numpy as jnp
import numpy as np
from jax import lax
from jax.experimental import pallas as pl
from jax.experimental.pallas import tpu as pltpu
from jax.experimental.pallas import tpu_sc as plsc

N = 10000
E = 320000
NS = 16
K = 80
BE = 1280

ROWS_PER_TILE = N // NS
E_PER_TILE = E // NS
NCHUNK = E_PER_TILE // K
ZROWS = 25

_SQRT3 = math.sqrt(3.0)
_INV_SQRT_NEIGH = 1.0 / math.sqrt(32.0)

_COLPERM = np.concatenate([
    96 + np.arange(32),
    128 + np.arange(32),
    160 + (np.arange(96) % 3) * 32 + np.arange(96) // 3,
    (np.arange(96) % 3) * 32 + np.arange(96) // 3,
]).astype(np.int32)


def _tc_body(r_ref, v_ref, w1, w2, w3, w4, wT, aux0_ref, aux1_ref):
    r = r_ref[...]
    v = v_ref[...]
    h = jax.nn.silu(jnp.dot(r, w1[...], preferred_element_type=jnp.float32))
    h = jax.nn.silu(jnp.dot(h, w2[...], preferred_element_type=jnp.float32))
    h = jax.nn.silu(jnp.dot(h, w3[...], preferred_element_type=jnp.float32))
    mix = jnp.dot(h, w4[...], preferred_element_type=jnp.float32) * _INV_SQRT_NEIGH
    a_s = mix[:, 0:32]
    a_t0 = mix[:, 32:64] * (1.0 / _SQRT3)
    a_v = mix[:, 64:96]
    a3 = jnp.dot(h, wT[...], preferred_element_type=jnp.float32) * _INV_SQRT_NEIGH
    def _hilo(x):
        hi = x.astype(jnp.bfloat16).astype(jnp.float32)
        return jnp.concatenate([hi, x - hi], axis=1)

    sq = _hilo(v * v)
    r2 = jnp.dot(sq, jnp.ones((6, 16), jnp.float32),
                 preferred_element_type=jnp.float32)
    vn = _hilo(v * (_SQRT3 / jnp.sqrt(r2 + 1e-12))[:, 0:3])
    col = lax.broadcasted_iota(jnp.int32, (3, 96), 1)
    row = lax.broadcasted_iota(jnp.int32, (3, 96), 0)
    bsel = (col // 32 == row).astype(jnp.float32)
    bsel2 = jnp.concatenate([bsel, bsel], axis=0)
    yb = jnp.dot(vn, bsel2, preferred_element_type=jnp.float32)
    aux0_ref[:, 0:96] = yb * a3
    aux0_ref[:, 96:128] = a_s
    aux1_ref[:, 0:32] = a_v
    aux1_ref[:, 32:64] = a_t0
    aux1_ref[:, 64:80] = yb[:, 0:16]
    aux1_ref[:, 80:96] = yb[:, 32:48]
    aux1_ref[:, 96:112] = yb[:, 64:80]
    aux1_ref[:, 112:128] = jnp.zeros((r.shape[0], 16), jnp.float32)


_tc_aux = pl.pallas_call(
    _tc_body,
    grid=(E // BE,),
    in_specs=[
        pl.BlockSpec((BE, 8), lambda i: (i, 0)),
        pl.BlockSpec((BE, 3), lambda i: (i, 0)),
        pl.BlockSpec((8, 64), lambda i: (0, 0)),
        pl.BlockSpec((64, 64), lambda i: (0, 0)),
        pl.BlockSpec((64, 64), lambda i: (0, 0)),
        pl.BlockSpec((64, 128), lambda i: (0, 0)),
        pl.BlockSpec((64, 96), lambda i: (0, 0)),
    ],
    out_specs=[
        pl.BlockSpec((BE, 128), lambda i: (i, 0)),
        pl.BlockSpec((BE, 128), lambda i: (i, 0)),
    ],
    out_shape=[
        jax.ShapeDtypeStruct((E, 128), jnp.float32),
        jax.ShapeDtypeStruct((E, 128), jnp.float32),
    ],
)


def _sc_body(nfs, nfv, snd, rcv, aux0, aux1, out0, out1, acc, sidx, ridx,
             gs, gv, ab0, ab1, msg, zbuf, sem):
    c = lax.axis_index("c")
    s = lax.axis_index("s")
    r0 = s * ROWS_PER_TILE
    e0 = s * E_PER_TILE

    zeros16 = jnp.zeros((16,), jnp.float32)

    def _zrow(i, carry):
        for t in range(8):
            zbuf[i, pl.ds(16 * t, 16)] = zeros16
        return carry
    lax.fori_loop(0, ZROWS, _zrow, 0)
    for rep in range(ROWS_PER_TILE // ZROWS):
        pltpu.sync_copy(zbuf, acc.at[pl.ds(r0 + rep * ZROWS, ZROWS)])
    plsc.subcore_barrier()

    @pl.when(c == 0)
    def _core0():
        def chunk(j, carry):
            base = e0 + j * K
            pltpu.sync_copy(snd.at[pl.ds(base, K)], sidx)
            pltpu.sync_copy(rcv.at[pl.ds(base, K)], ridx)
            cp = pltpu.async_copy(nfs.at[sidx], gs, sem)
            pltpu.sync_copy(aux0.at[pl.ds(base, K)], ab0)
            cp.wait()

            def edge(k, ecarry):
                g0 = gs[k, pl.ds(0, 16)]
                g1 = gs[k, pl.ds(16, 16)]
                for t in range(6):
                    gh = g0 if t % 2 == 0 else g1
                    msg[k, pl.ds(16 * t, 16)] = gh * ab0[k, pl.ds(16 * t, 16)]
                msg[k, pl.ds(96, 16)] = g0 * ab0[k, pl.ds(96, 16)]
                msg[k, pl.ds(112, 16)] = g1 * ab0[k, pl.ds(112, 16)]
                return ecarry
            lax.fori_loop(0, K, edge, 0)
            pltpu.sync_copy(msg, acc.at[ridx], add=True)
            return carry
        lax.fori_loop(0, NCHUNK, chunk, 0)
        plsc.subcore_barrier()
        pltpu.sync_copy(acc.at[pl.ds(r0, ROWS_PER_TILE)],
                        out0.at[pl.ds(r0, ROWS_PER_TILE)])

    @pl.when(c == 1)
    def _core1():
        def chunk(j, carry):
            base = e0 + j * K
            pltpu.sync_copy(snd.at[pl.ds(base, K)], sidx)
            pltpu.sync_copy(rcv.at[pl.ds(base, K)], ridx)
            cp = pltpu.async_copy(nfv.at[sidx], gv, sem)
            pltpu.sync_copy(aux1.at[pl.ds(base, K)], ab1)
            cp.wait()

            def edge(k, ecarry):
                av0 = ab1[k, pl.ds(0, 16)]
                av1 = ab1[k, pl.ds(16, 16)]
                at0 = ab1[k, pl.ds(32, 16)]
                at1 = ab1[k, pl.ds(48, 16)]
                yb0 = ab1[k, pl.ds(64, 16)]
                yb1 = ab1[k, pl.ds(80, 16)]
                yb2 = ab1[k, pl.ds(96, 16)]
                gx0 = gv[k, pl.ds(0, 16)]
                gx1 = gv[k, pl.ds(16, 16)]
                gy0 = gv[k, pl.ds(32, 16)]
                gy1 = gv[k, pl.ds(48, 16)]
                gz0 = gv[k, pl.ds(64, 16)]
                gz1 = gv[k, pl.ds(80, 16)]
                msg[k, pl.ds(0, 16)] = (gx0 * yb0 + gy0 * yb1 + gz0 * yb2) * at0
                msg[k, pl.ds(16, 16)] = (gx1 * yb0 + gy1 * yb1 + gz1 * yb2) * at1
                msg[k, pl.ds(32, 16)] = gx0 * av0
                msg[k, pl.ds(48, 16)] = gx1 * av1
                msg[k, pl.ds(64, 16)] = gy0 * av0
                msg[k, pl.ds(80, 16)] = gy1 * av1
                msg[k, pl.ds(96, 16)] = gz0 * av0
                msg[k, pl.ds(112, 16)] = gz1 * av1
                return ecarry
            lax.fori_loop(0, K, edge, 0)
            pltpu.sync_copy(msg, acc.at[ridx], add=True)
            return carry
        lax.fori_loop(0, NCHUNK, chunk, 0)
        plsc.subcore_barrier()
        pltpu.sync_copy(acc.at[pl.ds(r0, ROWS_PER_TILE)],
                        out1.at[pl.ds(r0, ROWS_PER_TILE)])


def _make_sc():
    mesh = plsc.VectorSubcoreMesh(core_axis_name="c", subcore_axis_name="s")
    return pl.kernel(
        _sc_body,
        out_type=[jax.ShapeDtypeStruct((N, 128), jnp.float32),
                  jax.ShapeDtypeStruct((N, 128), jnp.float32)],
        mesh=mesh,
        compiler_params=pltpu.CompilerParams(use_tc_tiling_on_sc=False),
        scratch_types=[
            pltpu.VMEM_SHARED((N, 128), jnp.float32),
            pltpu.VMEM((K,), jnp.int32),
            pltpu.VMEM((K,), jnp.int32),
            pltpu.VMEM((K, 32), jnp.float32),
            pltpu.VMEM((K, 96), jnp.float32),
            pltpu.VMEM((K, 128), jnp.float32),
            pltpu.VMEM((K, 128), jnp.float32),
            pltpu.VMEM((K, 128), jnp.float32),
            pltpu.VMEM((ZROWS, 128), jnp.float32),
            pltpu.SemaphoreType.DMA,
        ],
    )


def kernel(vectors, node_feats, radial_embedding, senders, receivers,
           W1, W2, W3, W4):
    w1 = W1 * (1.0 / math.sqrt(W1.shape[0]))
    w2 = W2 * (1.0 / math.sqrt(W2.shape[0]))
    w3 = W3 * (1.0 / math.sqrt(W3.shape[0]))
    w4 = W4 * (1.0 / math.sqrt(W4.shape[0]))
    wT = jnp.concatenate([w4[:, 96:128]] * 3, axis=1)
    aux0, aux1 = _tc_aux(radial_embedding, vectors, w1, w2, w3, w4, wT)
    nf_s = node_feats[:, 0:32]
    nf_v = node_feats[:, 32:128].reshape(N, 32, 3).transpose(0, 2, 1).reshape(N, 96)
    snd = senders.astype(jnp.int32)
    rcv = receivers.astype(jnp.int32)
    out0, out1 = _make_sc()(nf_s, nf_v, snd, rcv, aux0, aux1)
    outp = jnp.concatenate([out0, out1], axis=1)
    return jnp.take(outp, jnp.asarray(_COLPERM), axis=1)


# speedup vs baseline: 3.1562x; 1.2166x over previous
import math

import jax
import jax.numpy as jnp
import numpy as np
from jax import lax
from jax.experimental import pallas as pl
from jax.experimental.pallas import tpu as pltpu
from jax.experimental.pallas import tpu_sc as plsc

N = 10000
E = 320000
NS = 16
K = 40
BE = 1280

ROWS_PER_TILE = N // NS
E_PER_TILE = E // NS
NCHUNK = E_PER_TILE // K
NPAIR = NCHUNK // 2
ZROWS = 25

_SQRT3 = math.sqrt(3.0)
_INV_SQRT_NEIGH = 1.0 / math.sqrt(32.0)

_COLPERM = np.concatenate([
    96 + np.arange(32),
    128 + np.arange(32),
    160 + (np.arange(96) % 3) * 32 + np.arange(96) // 3,
    (np.arange(96) % 3) * 32 + np.arange(96) // 3,
]).astype(np.int32)


def _tc_body(r_ref, v_ref, w1, w2, w3, w4, wT, aux0_ref, aux1_ref):
    r = r_ref[...]
    v = v_ref[...]
    h = jax.nn.silu(jnp.dot(r, w1[...], preferred_element_type=jnp.float32))
    h = jax.nn.silu(jnp.dot(h, w2[...], preferred_element_type=jnp.float32))
    h = jax.nn.silu(jnp.dot(h, w3[...], preferred_element_type=jnp.float32))
    mix = jnp.dot(h, w4[...], preferred_element_type=jnp.float32) * _INV_SQRT_NEIGH
    a_s = mix[:, 0:32]
    a_t0 = mix[:, 32:64] * (1.0 / _SQRT3)
    a_v = mix[:, 64:96]
    a3 = jnp.dot(h, wT[...], preferred_element_type=jnp.float32) * _INV_SQRT_NEIGH
    def _hilo(x):
        hi = x.astype(jnp.bfloat16).astype(jnp.float32)
        return jnp.concatenate([hi, x - hi], axis=1)

    sq = _hilo(v * v)
    r2 = jnp.dot(sq, jnp.ones((6, 16), jnp.float32),
                 preferred_element_type=jnp.float32)
    vn = _hilo(v * (_SQRT3 / jnp.sqrt(r2 + 1e-12))[:, 0:3])
    col = lax.broadcasted_iota(jnp.int32, (3, 96), 1)
    row = lax.broadcasted_iota(jnp.int32, (3, 96), 0)
    bsel = (col // 32 == row).astype(jnp.float32)
    bsel2 = jnp.concatenate([bsel, bsel], axis=0)
    yb = jnp.dot(vn, bsel2, preferred_element_type=jnp.float32)
    aux0_ref[:, 0:96] = yb * a3
    aux0_ref[:, 96:128] = a_s
    aux1_ref[:, 0:32] = a_v
    aux1_ref[:, 32:64] = a_t0
    aux1_ref[:, 64:80] = yb[:, 0:16]
    aux1_ref[:, 80:96] = yb[:, 32:48]
    aux1_ref[:, 96:112] = yb[:, 64:80]
    aux1_ref[:, 112:128] = jnp.zeros((r.shape[0], 16), jnp.float32)


_tc_aux = pl.pallas_call(
    _tc_body,
    grid=(E // BE,),
    in_specs=[
        pl.BlockSpec((BE, 8), lambda i: (i, 0)),
        pl.BlockSpec((BE, 3), lambda i: (i, 0)),
        pl.BlockSpec((8, 64), lambda i: (0, 0)),
        pl.BlockSpec((64, 64), lambda i: (0, 0)),
        pl.BlockSpec((64, 64), lambda i: (0, 0)),
        pl.BlockSpec((64, 128), lambda i: (0, 0)),
        pl.BlockSpec((64, 96), lambda i: (0, 0)),
    ],
    out_specs=[
        pl.BlockSpec((BE, 128), lambda i: (i, 0)),
        pl.BlockSpec((BE, 128), lambda i: (i, 0)),
    ],
    out_shape=[
        jax.ShapeDtypeStruct((E, 128), jnp.float32),
        jax.ShapeDtypeStruct((E, 128), jnp.float32),
    ],
)


def _sc_body(nfs, nfv, snd, rcv, aux0, aux1, out0, out1, acc,
             sidx_a, sidx_b, ridx_a, ridx_b, gs_a, gs_b, gv_a, gv_b,
             ab_a, ab_b, msg, zbuf, sem_g, sem_a, sem_i):
    c = lax.axis_index("c")
    s = lax.axis_index("s")
    r0 = s * ROWS_PER_TILE
    e0 = s * E_PER_TILE

    zeros16 = jnp.zeros((16,), jnp.float32)

    def _zrow(i, carry):
        for t in range(8):
            zbuf[i, pl.ds(16 * t, 16)] = zeros16
        return carry
    lax.fori_loop(0, ZROWS, _zrow, 0)
    for rep in range(ROWS_PER_TILE // ZROWS):
        pltpu.sync_copy(zbuf, acc.at[pl.ds(r0 + rep * ZROWS, ZROWS)])
    plsc.subcore_barrier()

    def run(nf, aux, gbufs, out, edgefn):
        sidx = (sidx_a, sidx_b)
        ridx = (ridx_a, ridx_b)
        ab = (ab_a, ab_b)

        pltpu.sync_copy(snd.at[pl.ds(e0, K)], sidx_a)
        pltpu.sync_copy(rcv.at[pl.ds(e0, K)], ridx_a)
        pltpu.async_copy(nf.at[sidx_a], gbufs[0], sem_g)
        pltpu.async_copy(aux.at[pl.ds(e0, K)], ab_a, sem_a)
        pltpu.async_copy(snd.at[pl.ds(e0 + K, K)], sidx_b, sem_i)
        pltpu.async_copy(rcv.at[pl.ds(e0 + K, K)], ridx_b, sem_i)

        def pair(jp, carry):
            for b in (0, 1):
                j = 2 * jp + b
                p, q = b, 1 - b
                last = jp == NPAIR - 1
                nbase = e0 + (j + 1) * K

                def _launch_next():
                    pltpu.make_async_copy(
                        snd.at[pl.ds(nbase, K)], sidx[q], sem_i).wait()
                    pltpu.make_async_copy(
                        rcv.at[pl.ds(nbase, K)], ridx[q], sem_i).wait()
                    pltpu.async_copy(nf.at[sidx[q]], gbufs[q], sem_g)
                    pltpu.async_copy(aux.at[pl.ds(nbase, K)], ab[q], sem_a)
                if b == 0:
                    _launch_next()
                else:
                    pl.when(jnp.logical_not(last))(_launch_next)

                pltpu.make_async_copy(
                    nf.at[pl.ds(0, K)], gbufs[p], sem_g).wait()
                pltpu.make_async_copy(
                    aux.at[pl.ds(e0 + j * K, K)], ab[p], sem_a).wait()
                edgefn(gbufs[p], ab[p])
                pltpu.sync_copy(msg, acc.at[ridx[p]], add=True)

                @pl.when(jnp.logical_not(last))
                def _prefetch_idx():
                    base2 = e0 + (j + 2) * K
                    pltpu.async_copy(snd.at[pl.ds(base2, K)], sidx[p], sem_i)
                    pltpu.async_copy(rcv.at[pl.ds(base2, K)], ridx[p], sem_i)
            return carry
        lax.fori_loop(0, NPAIR, pair, 0)
        plsc.subcore_barrier()
        pltpu.sync_copy(acc.at[pl.ds(r0, ROWS_PER_TILE)],
                        out.at[pl.ds(r0, ROWS_PER_TILE)])

    @pl.when(c == 0)
    def _core0():
        def edge0(gs, ab0):
            def edge(k, ecarry):
                g0 = gs[k, pl.ds(0, 16)]
                g1 = gs[k, pl.ds(16, 16)]
                for t in range(6):
                    gh = g0 if t % 2 == 0 else g1
                    msg[k, pl.ds(16 * t, 16)] = gh * ab0[k, pl.ds(16 * t, 16)]
                msg[k, pl.ds(96, 16)] = g0 * ab0[k, pl.ds(96, 16)]
                msg[k, pl.ds(112, 16)] = g1 * ab0[k, pl.ds(112, 16)]
                return ecarry
            lax.fori_loop(0, K, edge, 0)
        run(nfs, aux0, (gs_a, gs_b), out0, edge0)

    @pl.when(c == 1)
    def _core1():
        def edge1(gv, ab1):
            def edge(k, ecarry):
                av0 = ab1[k, pl.ds(0, 16)]
                av1 = ab1[k, pl.ds(16, 16)]
                at0 = ab1[k, pl.ds(32, 16)]
                at1 = ab1[k, pl.ds(48, 16)]
                yb0 = ab1[k, pl.ds(64, 16)]
                yb1 = ab1[k, pl.ds(80, 16)]
                yb2 = ab1[k, pl.ds(96, 16)]
                gx0 = gv[k, pl.ds(0, 16)]
                gx1 = gv[k, pl.ds(16, 16)]
                gy0 = gv[k, pl.ds(32, 16)]
                gy1 = gv[k, pl.ds(48, 16)]
                gz0 = gv[k, pl.ds(64, 16)]
                gz1 = gv[k, pl.ds(80, 16)]
                msg[k, pl.ds(0, 16)] = (gx0 * yb0 + gy0 * yb1 + gz0 * yb2) * at0
                msg[k, pl.ds(16, 16)] = (gx1 * yb0 + gy1 * yb1 + gz1 * yb2) * at1
                msg[k, pl.ds(32, 16)] = gx0 * av0
                msg[k, pl.ds(48, 16)] = gx1 * av1
                msg[k, pl.ds(64, 16)] = gy0 * av0
                msg[k, pl.ds(80, 16)] = gy1 * av1
                msg[k, pl.ds(96, 16)] = gz0 * av0
                msg[k, pl.ds(112, 16)] = gz1 * av1
                return ecarry
            lax.fori_loop(0, K, edge, 0)
        run(nfv, aux1, (gv_a, gv_b), out1, edge1)


def _make_sc():
    mesh = plsc.VectorSubcoreMesh(core_axis_name="c", subcore_axis_name="s")
    return pl.kernel(
        _sc_body,
        out_type=[jax.ShapeDtypeStruct((N, 128), jnp.float32),
                  jax.ShapeDtypeStruct((N, 128), jnp.float32)],
        mesh=mesh,
        compiler_params=pltpu.CompilerParams(use_tc_tiling_on_sc=False),
        scratch_types=[
            pltpu.VMEM_SHARED((N, 128), jnp.float32),
            pltpu.VMEM((K,), jnp.int32),
            pltpu.VMEM((K,), jnp.int32),
            pltpu.VMEM((K,), jnp.int32),
            pltpu.VMEM((K,), jnp.int32),
            pltpu.VMEM((K, 32), jnp.float32),
            pltpu.VMEM((K, 32), jnp.float32),
            pltpu.VMEM((K, 96), jnp.float32),
            pltpu.VMEM((K, 96), jnp.float32),
            pltpu.VMEM((K, 128), jnp.float32),
            pltpu.VMEM((K, 128), jnp.float32),
            pltpu.VMEM((K, 128), jnp.float32),
            pltpu.VMEM((ZROWS, 128), jnp.float32),
            pltpu.SemaphoreType.DMA,
            pltpu.SemaphoreType.DMA,
            pltpu.SemaphoreType.DMA,
        ],
    )


def kernel(vectors, node_feats, radial_embedding, senders, receivers,
           W1, W2, W3, W4):
    w1 = W1 * (1.0 / math.sqrt(W1.shape[0]))
    w2 = W2 * (1.0 / math.sqrt(W2.shape[0]))
    w3 = W3 * (1.0 / math.sqrt(W3.shape[0]))
    w4 = W4 * (1.0 / math.sqrt(W4.shape[0]))
    wT = jnp.concatenate([w4[:, 96:128]] * 3, axis=1)
    aux0, aux1 = _tc_aux(radial_embedding, vectors, w1, w2, w3, w4, wT)
    nf_s = node_feats[:, 0:32]
    nf_v = node_feats[:, 32:128].reshape(N, 32, 3).transpose(0, 2, 1).reshape(N, 96)
    snd = senders.astype(jnp.int32)
    rcv = receivers.astype(jnp.int32)
    out0, out1 = _make_sc()(nf_s, nf_v, snd, rcv, aux0, aux1)
    outp = jnp.concatenate([out0, out1], axis=1)
    return jnp.take(outp, jnp.asarray(_COLPERM), axis=1)


# speedup vs baseline: 3.9909x; 1.2644x over previous
import math

import jax
import jax.numpy as jnp
import numpy as np
from jax import lax
from jax.experimental import pallas as pl
from jax.experimental.pallas import tpu as pltpu
from jax.experimental.pallas import tpu_sc as plsc

N = 10000
E = 320000
NS = 16
K = 40
BE = 1280
EH = E // 2

ROWS_PER_TILE = N // NS
E_PER_TILE = EH // NS
NCHUNK = E_PER_TILE // K
NPAIR = NCHUNK // 2
ZROWS = 25

_SQRT3 = math.sqrt(3.0)
_INV_SQRT_NEIGH = 1.0 / math.sqrt(32.0)

_NFPERM = np.concatenate([
    np.arange(32),
    32 + (np.arange(96) // 32) + 3 * (np.arange(96) % 32),
]).astype(np.int32)

_COLPERM = np.concatenate([
    96 + np.arange(32),
    128 + np.arange(32),
    160 + (np.arange(96) % 3) * 32 + np.arange(96) // 3,
    (np.arange(96) % 3) * 32 + np.arange(96) // 3,
]).astype(np.int32)


def _tc_body(r_ref, v_ref, w1, w2, w3, w4, wT, aux0_ref, aux1_ref):
    r = r_ref[...]
    v = v_ref[...]
    h = jax.nn.silu(jnp.dot(r, w1[...], preferred_element_type=jnp.float32))
    h = jax.nn.silu(jnp.dot(h, w2[...], preferred_element_type=jnp.float32))
    h = jax.nn.silu(jnp.dot(h, w3[...], preferred_element_type=jnp.float32))
    mix = jnp.dot(h, w4[...], preferred_element_type=jnp.float32) * _INV_SQRT_NEIGH
    a_s = mix[:, 0:32]
    a_t0 = mix[:, 32:64] * (1.0 / _SQRT3)
    a_v = mix[:, 64:96]
    a3 = jnp.dot(h, wT[...], preferred_element_type=jnp.float32) * _INV_SQRT_NEIGH
    def _hilo(x):
        hi = x.astype(jnp.bfloat16).astype(jnp.float32)
        return jnp.concatenate([hi, x - hi], axis=1)

    sq = _hilo(v * v)
    r2 = jnp.dot(sq, jnp.ones((6, 16), jnp.float32),
                 preferred_element_type=jnp.float32)
    vn = _hilo(v * (_SQRT3 / jnp.sqrt(r2 + 1e-12))[:, 0:3])
    col = lax.broadcasted_iota(jnp.int32, (3, 96), 1)
    row = lax.broadcasted_iota(jnp.int32, (3, 96), 0)
    bsel = (col // 32 == row).astype(jnp.float32)
    bsel2 = jnp.concatenate([bsel, bsel], axis=0)
    yb = jnp.dot(vn, bsel2, preferred_element_type=jnp.float32)
    aux0_ref[:, 0:96] = yb * a3
    aux0_ref[:, 96:128] = a_s
    aux1_ref[:, 0:32] = a_v
    aux1_ref[:, 32:64] = a_t0
    aux1_ref[:, 64:80] = yb[:, 0:16]
    aux1_ref[:, 80:96] = yb[:, 32:48]
    aux1_ref[:, 96:112] = yb[:, 64:80]
    aux1_ref[:, 112:128] = jnp.zeros((r.shape[0], 16), jnp.float32)


_tc_aux = pl.pallas_call(
    _tc_body,
    grid=(EH // BE,),
    in_specs=[
        pl.BlockSpec((BE, 8), lambda i: (i, 0)),
        pl.BlockSpec((BE, 3), lambda i: (i, 0)),
        pl.BlockSpec((8, 64), lambda i: (0, 0)),
        pl.BlockSpec((64, 64), lambda i: (0, 0)),
        pl.BlockSpec((64, 64), lambda i: (0, 0)),
        pl.BlockSpec((64, 128), lambda i: (0, 0)),
        pl.BlockSpec((64, 96), lambda i: (0, 0)),
    ],
    out_specs=[
        pl.BlockSpec((BE, 128), lambda i: (i, 0)),
        pl.BlockSpec((BE, 128), lambda i: (i, 0)),
    ],
    out_shape=[
        jax.ShapeDtypeStruct((EH, 128), jnp.float32),
        jax.ShapeDtypeStruct((EH, 128), jnp.float32),
    ],
)


def _sc_body(nfp, snd, rcv, aux0, aux1, out0, out1, acc,
             sidx_a, sidx_b, ridx_a, ridx_b, g_a, g_b,
             ab_a, ab_b, msg, zbuf, sem_g, sem_a, sem_i):
    c = lax.axis_index("c")
    s = lax.axis_index("s")
    r0 = s * ROWS_PER_TILE
    e0 = s * E_PER_TILE

    zeros16 = jnp.zeros((16,), jnp.float32)

    def _zrow(i, carry):
        for t in range(8):
            zbuf[i, pl.ds(16 * t, 16)] = zeros16
        return carry
    lax.fori_loop(0, ZROWS, _zrow, 0)
    for rep in range(ROWS_PER_TILE // ZROWS):
        pltpu.sync_copy(zbuf, acc.at[pl.ds(r0 + rep * ZROWS, ZROWS)])
    plsc.subcore_barrier()

    def run(aux, out, edgefn):
        nf = nfp
        gbufs = (g_a, g_b)
        sidx = (sidx_a, sidx_b)
        ridx = (ridx_a, ridx_b)
        ab = (ab_a, ab_b)

        pltpu.sync_copy(snd.at[pl.ds(e0, K)], sidx_a)
        pltpu.sync_copy(rcv.at[pl.ds(e0, K)], ridx_a)
        pltpu.async_copy(nf.at[sidx_a], gbufs[0], sem_g)
        pltpu.async_copy(aux.at[pl.ds(e0, K)], ab_a, sem_a)
        pltpu.async_copy(snd.at[pl.ds(e0 + K, K)], sidx_b, sem_i)
        pltpu.async_copy(rcv.at[pl.ds(e0 + K, K)], ridx_b, sem_i)

        def pair(jp, carry):
            for b in (0, 1):
                j = 2 * jp + b
                p, q = b, 1 - b
                last = jp == NPAIR - 1
                nbase = e0 + (j + 1) * K

                def _launch_next():
                    pltpu.make_async_copy(
                        snd.at[pl.ds(nbase, K)], sidx[q], sem_i).wait()
                    pltpu.make_async_copy(
                        rcv.at[pl.ds(nbase, K)], ridx[q], sem_i).wait()
                    pltpu.async_copy(nf.at[sidx[q]], gbufs[q], sem_g)
                    pltpu.async_copy(aux.at[pl.ds(nbase, K)], ab[q], sem_a)
                if b == 0:
                    _launch_next()
                else:
                    pl.when(jnp.logical_not(last))(_launch_next)

                pltpu.make_async_copy(
                    nf.at[pl.ds(0, K)], gbufs[p], sem_g).wait()
                pltpu.make_async_copy(
                    aux.at[pl.ds(e0 + j * K, K)], ab[p], sem_a).wait()
                edgefn(gbufs[p], ab[p])
                pltpu.sync_copy(msg, acc.at[ridx[p]], add=True)

                @pl.when(jnp.logical_not(last))
                def _prefetch_idx():
                    base2 = e0 + (j + 2) * K
                    pltpu.async_copy(snd.at[pl.ds(base2, K)], sidx[p], sem_i)
                    pltpu.async_copy(rcv.at[pl.ds(base2, K)], ridx[p], sem_i)
            return carry
        lax.fori_loop(0, NPAIR, pair, 0)
        plsc.subcore_barrier()
        pltpu.sync_copy(acc.at[pl.ds(r0, ROWS_PER_TILE)],
                        out.at[pl.ds(r0, ROWS_PER_TILE)])

    @pl.when(c == 0)
    def _core0():
        def edge0(gs, ab0):
            def edge(k, ecarry):
                g0 = gs[k, pl.ds(0, 16)]
                g1 = gs[k, pl.ds(16, 16)]
                for t in range(6):
                    gh = g0 if t % 2 == 0 else g1
                    msg[k, pl.ds(16 * t, 16)] = gh * ab0[k, pl.ds(16 * t, 16)]
                msg[k, pl.ds(96, 16)] = g0 * ab0[k, pl.ds(96, 16)]
                msg[k, pl.ds(112, 16)] = g1 * ab0[k, pl.ds(112, 16)]
                return ecarry
            lax.fori_loop(0, K, edge, 0)
        run(aux0, out0, edge0)

    @pl.when(c == 1)
    def _core1():
        def edge1(gv, ab1):
            def edge(k, ecarry):
                av0 = ab1[k, pl.ds(0, 16)]
                av1 = ab1[k, pl.ds(16, 16)]
                at0 = ab1[k, pl.ds(32, 16)]
                at1 = ab1[k, pl.ds(48, 16)]
                yb0 = ab1[k, pl.ds(64, 16)]
                yb1 = ab1[k, pl.ds(80, 16)]
                yb2 = ab1[k, pl.ds(96, 16)]
                gx0 = gv[k, pl.ds(32, 16)]
                gx1 = gv[k, pl.ds(48, 16)]
                gy0 = gv[k, pl.ds(64, 16)]
                gy1 = gv[k, pl.ds(80, 16)]
                gz0 = gv[k, pl.ds(96, 16)]
                gz1 = gv[k, pl.ds(112, 16)]
                msg[k, pl.ds(0, 16)] = (gx0 * yb0 + gy0 * yb1 + gz0 * yb2) * at0
                msg[k, pl.ds(16, 16)] = (gx1 * yb0 + gy1 * yb1 + gz1 * yb2) * at1
                msg[k, pl.ds(32, 16)] = gx0 * av0
                msg[k, pl.ds(48, 16)] = gx1 * av1
                msg[k, pl.ds(64, 16)] = gy0 * av0
                msg[k, pl.ds(80, 16)] = gy1 * av1
                msg[k, pl.ds(96, 16)] = gz0 * av0
                msg[k, pl.ds(112, 16)] = gz1 * av1
                return ecarry
            lax.fori_loop(0, K, edge, 0)
        run(aux1, out1, edge1)


def _make_sc():
    mesh = plsc.VectorSubcoreMesh(core_axis_name="c", subcore_axis_name="s")
    return pl.kernel(
        _sc_body,
        out_type=[jax.ShapeDtypeStruct((N, 128), jnp.float32),
                  jax.ShapeDtypeStruct((N, 128), jnp.float32)],
        mesh=mesh,
        compiler_params=pltpu.CompilerParams(use_tc_tiling_on_sc=False),
        scratch_types=[
            pltpu.VMEM_SHARED((N, 128), jnp.float32),
            pltpu.VMEM((K,), jnp.int32),
            pltpu.VMEM((K,), jnp.int32),
            pltpu.VMEM((K,), jnp.int32),
            pltpu.VMEM((K,), jnp.int32),
            pltpu.VMEM((K, 128), jnp.float32),
            pltpu.VMEM((K, 128), jnp.float32),
            pltpu.VMEM((K, 128), jnp.float32),
            pltpu.VMEM((K, 128), jnp.float32),
            pltpu.VMEM((K, 128), jnp.float32),
            pltpu.VMEM((ZROWS, 128), jnp.float32),
            pltpu.SemaphoreType.DMA,
            pltpu.SemaphoreType.DMA,
            pltpu.SemaphoreType.DMA,
        ],
    )


def kernel(vectors, node_feats, radial_embedding, senders, receivers,
           W1, W2, W3, W4):
    w1 = W1 * (1.0 / math.sqrt(W1.shape[0]))
    w2 = W2 * (1.0 / math.sqrt(W2.shape[0]))
    w3 = W3 * (1.0 / math.sqrt(W3.shape[0]))
    w4 = W4 * (1.0 / math.sqrt(W4.shape[0]))
    wT = jnp.concatenate([w4[:, 96:128]] * 3, axis=1)
    nfp = jnp.take(node_feats, jnp.asarray(_NFPERM), axis=1)
    snd = senders.astype(jnp.int32)
    rcv = receivers.astype(jnp.int32)
    sc = _make_sc()
    outs = []
    for lo in (0, EH):
        aux0, aux1 = _tc_aux(radial_embedding[lo:lo + EH],
                             vectors[lo:lo + EH], w1, w2, w3, w4, wT)
        outs.append(sc(nfp, snd[lo:lo + EH], rcv[lo:lo + EH], aux0, aux1))
    out0 = outs[0][0] + outs[1][0]
    out1 = outs[0][1] + outs[1][1]
    outp = jnp.concatenate([out0, out1], axis=1)
    return jnp.take(outp, jnp.asarray(_COLPERM), axis=1)


# speedup vs baseline: 4.1733x; 1.0457x over previous
import math

import jax
import jax.numpy as jnp
import numpy as np
from jax import lax
from jax.experimental import pallas as pl
from jax.experimental.pallas import tpu as pltpu
from jax.experimental.pallas import tpu_sc as plsc

N = 10000
E = 320000
NS = 16
K = 40
BE = 1000
NSLAB = 4
EH = E // NSLAB

ROWS_PER_TILE = N // NS
E_PER_TILE = EH // NS
NCHUNK = E_PER_TILE // K
NPAIR = (NCHUNK - 1) // 2
ZROWS = 25

_SQRT3 = math.sqrt(3.0)
_INV_SQRT_NEIGH = 1.0 / math.sqrt(32.0)

_NFPERM = np.concatenate([
    np.arange(32),
    32 + (np.arange(96) // 32) + 3 * (np.arange(96) % 32),
]).astype(np.int32)

_COLPERM = np.concatenate([
    96 + np.arange(32),
    128 + np.arange(32),
    160 + (np.arange(96) % 3) * 32 + np.arange(96) // 3,
    (np.arange(96) % 3) * 32 + np.arange(96) // 3,
]).astype(np.int32)


def _tc_body(r_ref, v_ref, w1, w2, w3, w4, wT, aux0_ref, aux1_ref):
    r = r_ref[...]
    v = v_ref[...]
    h = jax.nn.silu(jnp.dot(r, w1[...], preferred_element_type=jnp.float32))
    h = jax.nn.silu(jnp.dot(h, w2[...], preferred_element_type=jnp.float32))
    h = jax.nn.silu(jnp.dot(h, w3[...], preferred_element_type=jnp.float32))
    mix = jnp.dot(h, w4[...], preferred_element_type=jnp.float32) * _INV_SQRT_NEIGH
    a_s = mix[:, 0:32]
    a_t0 = mix[:, 32:64] * (1.0 / _SQRT3)
    a_v = mix[:, 64:96]
    a3 = jnp.dot(h, wT[...], preferred_element_type=jnp.float32) * _INV_SQRT_NEIGH
    def _hilo(x):
        hi = x.astype(jnp.bfloat16).astype(jnp.float32)
        return jnp.concatenate([hi, x - hi], axis=1)

    sq = _hilo(v * v)
    r2 = jnp.dot(sq, jnp.ones((6, 16), jnp.float32),
                 preferred_element_type=jnp.float32)
    vn = _hilo(v * (_SQRT3 / jnp.sqrt(r2 + 1e-12))[:, 0:3])
    col = lax.broadcasted_iota(jnp.int32, (3, 96), 1)
    row = lax.broadcasted_iota(jnp.int32, (3, 96), 0)
    bsel = (col // 32 == row).astype(jnp.float32)
    bsel2 = jnp.concatenate([bsel, bsel], axis=0)
    yb = jnp.dot(vn, bsel2, preferred_element_type=jnp.float32)
    aux0_ref[:, 0:96] = yb * a3
    aux0_ref[:, 96:128] = a_s
    aux1_ref[:, 0:32] = a_v
    aux1_ref[:, 32:64] = a_t0
    aux1_ref[:, 64:80] = yb[:, 0:16]
    aux1_ref[:, 80:96] = yb[:, 32:48]
    aux1_ref[:, 96:112] = yb[:, 64:80]
    aux1_ref[:, 112:128] = jnp.zeros((r.shape[0], 16), jnp.float32)


_tc_aux = pl.pallas_call(
    _tc_body,
    grid=(EH // BE,),
    in_specs=[
        pl.BlockSpec((BE, 8), lambda i: (i, 0)),
        pl.BlockSpec((BE, 3), lambda i: (i, 0)),
        pl.BlockSpec((8, 64), lambda i: (0, 0)),
        pl.BlockSpec((64, 64), lambda i: (0, 0)),
        pl.BlockSpec((64, 64), lambda i: (0, 0)),
        pl.BlockSpec((64, 128), lambda i: (0, 0)),
        pl.BlockSpec((64, 96), lambda i: (0, 0)),
    ],
    out_specs=[
        pl.BlockSpec((BE, 128), lambda i: (i, 0)),
        pl.BlockSpec((BE, 128), lambda i: (i, 0)),
    ],
    out_shape=[
        jax.ShapeDtypeStruct((EH, 128), jnp.float32),
        jax.ShapeDtypeStruct((EH, 128), jnp.float32),
    ],
)


def _sc_body(nfp, snd, rcv, aux0, aux1, out0, out1, acc,
             sidx_a, sidx_b, ridx_a, ridx_b, g_a, g_b,
             ab_a, ab_b, msg, zbuf, sem_g, sem_a, sem_i):
    c = lax.axis_index("c")
    s = lax.axis_index("s")
    r0 = s * ROWS_PER_TILE
    e0 = s * E_PER_TILE

    zeros16 = jnp.zeros((16,), jnp.float32)

    def _zrow(i, carry):
        for t in range(8):
            zbuf[i, pl.ds(16 * t, 16)] = zeros16
        return carry
    lax.fori_loop(0, ZROWS, _zrow, 0)
    for rep in range(ROWS_PER_TILE // ZROWS):
        pltpu.sync_copy(zbuf, acc.at[pl.ds(r0 + rep * ZROWS, ZROWS)])
    plsc.subcore_barrier()

    def run(aux, out, edgefn):
        nf = nfp
        gbufs = (g_a, g_b)
        sidx = (sidx_a, sidx_b)
        ridx = (ridx_a, ridx_b)
        ab = (ab_a, ab_b)

        pltpu.sync_copy(snd.at[pl.ds(e0, K)], sidx_a)
        pltpu.sync_copy(rcv.at[pl.ds(e0, K)], ridx_a)
        pltpu.async_copy(nf.at[sidx_a], gbufs[0], sem_g)
        pltpu.async_copy(aux.at[pl.ds(e0, K)], ab_a, sem_a)
        pltpu.async_copy(snd.at[pl.ds(e0 + K, K)], sidx_b, sem_i)
        pltpu.async_copy(rcv.at[pl.ds(e0 + K, K)], ridx_b, sem_i)

        def pair(jp, carry):
            for b in (0, 1):
                j = 2 * jp + b
                p, q = b, 1 - b
                nbase = e0 + (j + 1) * K

                pltpu.make_async_copy(
                    snd.at[pl.ds(nbase, K)], sidx[q], sem_i).wait()
                pltpu.make_async_copy(
                    rcv.at[pl.ds(nbase, K)], ridx[q], sem_i).wait()
                pltpu.async_copy(nf.at[sidx[q]], gbufs[q], sem_g)
                pltpu.async_copy(aux.at[pl.ds(nbase, K)], ab[q], sem_a)

                pltpu.make_async_copy(
                    nf.at[pl.ds(0, K)], gbufs[p], sem_g).wait()
                pltpu.make_async_copy(
                    aux.at[pl.ds(e0 + j * K, K)], ab[p], sem_a).wait()
                edgefn(gbufs[p], ab[p])
                pltpu.sync_copy(msg, acc.at[ridx[p]], add=True)

                def _prefetch_idx():
                    base2 = e0 + (j + 2) * K
                    pltpu.async_copy(snd.at[pl.ds(base2, K)], sidx[p], sem_i)
                    pltpu.async_copy(rcv.at[pl.ds(base2, K)], ridx[p], sem_i)
                if b == 0:
                    _prefetch_idx()
                else:
                    pl.when(jp < NPAIR - 1)(_prefetch_idx)
            return carry
        lax.fori_loop(0, NPAIR, pair, 0)

        jl = NCHUNK - 1
        pltpu.make_async_copy(nf.at[pl.ds(0, K)], gbufs[0], sem_g).wait()
        pltpu.make_async_copy(
            aux.at[pl.ds(e0 + jl * K, K)], ab[0], sem_a).wait()
        edgefn(gbufs[0], ab[0])
        pltpu.sync_copy(msg, acc.at[ridx[0]], add=True)
        plsc.subcore_barrier()
        pltpu.sync_copy(acc.at[pl.ds(r0, ROWS_PER_TILE)],
                        out.at[pl.ds(r0, ROWS_PER_TILE)])

    @pl.when(c == 0)
    def _core0():
        def edge0(gs, ab0):
            def edge(k, ecarry):
                g0 = gs[k, pl.ds(0, 16)]
                g1 = gs[k, pl.ds(16, 16)]
                for t in range(6):
                    gh = g0 if t % 2 == 0 else g1
                    msg[k, pl.ds(16 * t, 16)] = gh * ab0[k, pl.ds(16 * t, 16)]
                msg[k, pl.ds(96, 16)] = g0 * ab0[k, pl.ds(96, 16)]
                msg[k, pl.ds(112, 16)] = g1 * ab0[k, pl.ds(112, 16)]
                return ecarry
            lax.fori_loop(0, K, edge, 0)
        run(aux0, out0, edge0)

    @pl.when(c == 1)
    def _core1():
        def edge1(gv, ab1):
            def edge(k, ecarry):
                av0 = ab1[k, pl.ds(0, 16)]
                av1 = ab1[k, pl.ds(16, 16)]
                at0 = ab1[k, pl.ds(32, 16)]
                at1 = ab1[k, pl.ds(48, 16)]
                yb0 = ab1[k, pl.ds(64, 16)]
                yb1 = ab1[k, pl.ds(80, 16)]
                yb2 = ab1[k, pl.ds(96, 16)]
                gx0 = gv[k, pl.ds(32, 16)]
                gx1 = gv[k, pl.ds(48, 16)]
                gy0 = gv[k, pl.ds(64, 16)]
                gy1 = gv[k, pl.ds(80, 16)]
                gz0 = gv[k, pl.ds(96, 16)]
                gz1 = gv[k, pl.ds(112, 16)]
                msg[k, pl.ds(0, 16)] = (gx0 * yb0 + gy0 * yb1 + gz0 * yb2) * at0
                msg[k, pl.ds(16, 16)] = (gx1 * yb0 + gy1 * yb1 + gz1 * yb2) * at1
                msg[k, pl.ds(32, 16)] = gx0 * av0
                msg[k, pl.ds(48, 16)] = gx1 * av1
                msg[k, pl.ds(64, 16)] = gy0 * av0
                msg[k, pl.ds(80, 16)] = gy1 * av1
                msg[k, pl.ds(96, 16)] = gz0 * av0
                msg[k, pl.ds(112, 16)] = gz1 * av1
                return ecarry
            lax.fori_loop(0, K, edge, 0)
        run(aux1, out1, edge1)


def _make_sc():
    mesh = plsc.VectorSubcoreMesh(core_axis_name="c", subcore_axis_name="s")
    return pl.kernel(
        _sc_body,
        out_type=[jax.ShapeDtypeStruct((N, 128), jnp.float32),
                  jax.ShapeDtypeStruct((N, 128), jnp.float32)],
        mesh=mesh,
        compiler_params=pltpu.CompilerParams(use_tc_tiling_on_sc=False),
        scratch_types=[
            pltpu.VMEM_SHARED((N, 128), jnp.float32),
            pltpu.VMEM((K,), jnp.int32),
            pltpu.VMEM((K,), jnp.int32),
            pltpu.VMEM((K,), jnp.int32),
            pltpu.VMEM((K,), jnp.int32),
            pltpu.VMEM((K, 128), jnp.float32),
            pltpu.VMEM((K, 128), jnp.float32),
            pltpu.VMEM((K, 128), jnp.float32),
            pltpu.VMEM((K, 128), jnp.float32),
            pltpu.VMEM((K, 128), jnp.float32),
            pltpu.VMEM((ZROWS, 128), jnp.float32),
            pltpu.SemaphoreType.DMA,
            pltpu.SemaphoreType.DMA,
            pltpu.SemaphoreType.DMA,
        ],
    )


def kernel(vectors, node_feats, radial_embedding, senders, receivers,
           W1, W2, W3, W4):
    w1 = W1 * (1.0 / math.sqrt(W1.shape[0]))
    w2 = W2 * (1.0 / math.sqrt(W2.shape[0]))
    w3 = W3 * (1.0 / math.sqrt(W3.shape[0]))
    w4 = W4 * (1.0 / math.sqrt(W4.shape[0]))
    wT = jnp.concatenate([w4[:, 96:128]] * 3, axis=1)
    nfp = jnp.take(node_feats, jnp.asarray(_NFPERM), axis=1)
    snd = senders.astype(jnp.int32)
    rcv = receivers.astype(jnp.int32)
    sc = _make_sc()
    outs = []
    for i in range(NSLAB):
        lo = i * EH
        aux0, aux1 = _tc_aux(radial_embedding[lo:lo + EH],
                             vectors[lo:lo + EH], w1, w2, w3, w4, wT)
        outs.append(sc(nfp, snd[lo:lo + EH], rcv[lo:lo + EH], aux0, aux1))
    out0 = sum((o[0] for o in outs[1:]), outs[0][0])
    out1 = sum((o[1] for o in outs[1:]), outs[0][1])
    outp = jnp.concatenate([out0, out1], axis=1)
    return jnp.take(outp, jnp.asarray(_COLPERM), axis=1)
